# baseline (speedup 1.0000x reference)
import numpy as np
import jax
import jax.numpy as jnp
from jax.experimental import pallas as pl
from jax.experimental.pallas import tpu as pltpu

_N = 384
_H = 193
_HP = 208
_NCH = 13
_CH = 16
_K = 16
_S = 2
_BIG = np.int32(1 << 30)


def _dft_consts():
    j = np.arange(_N)
    m = np.outer(j, j) % _N
    ang = -2.0 * np.pi * m / _N
    fre = (np.cos(ang) / np.sqrt(_N)).astype(np.float32)
    fim = (np.sin(ang) / np.sqrt(_N)).astype(np.float32)
    fre_h = np.zeros((_HP, _N), np.float32)
    fim_h = np.zeros((_HP, _N), np.float32)
    fre_h[:_H] = fre[:_H]
    fim_h[:_H] = fim[:_H]
    import ml_dtypes
    bf = ml_dtypes.bfloat16
    return fre_h.astype(bf), fim_h.astype(bf), fre.astype(bf), fim.astype(bf)


_FRE_H, _FIM_H, _FRE, _FIM = _dft_consts()


def _fft_topk_kernel(x_ref, freh_ref, fimh_ref, fre_ref, fim_ref, out_ref,
                     *p_scrs):
    freh = freh_ref[...]
    fimh = fimh_ref[...]
    fre = fre_ref[...]
    fim = fim_ref[...]

    def dot(a, b):
        return jax.lax.dot(a, b, preferred_element_type=jnp.float32)

    neg = jnp.float32(-np.inf)
    g_iota = jax.lax.broadcasted_iota(jnp.int32, (16, _N), 0)
    lr_iota = jax.lax.broadcasted_iota(jnp.int32, (_CH, _N), 0)
    lane = jax.lax.broadcasted_iota(jnp.int32, (1, _K), 1).astype(jnp.float32)

    carry = []
    for s in range(_S):
        x = x_ref[s]
        bre = dot(freh, x)
        bim = dot(fimh, x)
        breb = bre.astype(jnp.bfloat16)
        bimb = bim.astype(jnp.bfloat16)
        yre = dot(breb, fre) - dot(bimb, fim)
        yim = dot(breb, fim) + dot(bimb, fre)
        p = yre * yre + yim * yim

        mrows = [jnp.max(p[g * _CH:(g + 1) * _CH], axis=0, keepdims=True)
                 for g in range(_NCH)]
        mrows.append(jnp.full((16 - _NCH, _N), neg, jnp.float32))
        msum = jnp.concatenate(mrows, axis=0)
        p_scrs[s][...] = p.reshape(_NCH, _CH, _N)
        carry.append((msum, jnp.zeros((1, _K), jnp.float32),
                      jnp.zeros((1, 1), jnp.float32)))

    def body(_, carry):
        new = []
        for s in range(_S):
            msum, out, cnt = carry[s]
            m = jnp.max(msum, axis=(0, 1), keepdims=True)
            gmask = msum == m
            g = jnp.min(jnp.where(gmask, g_iota, _BIG))
            chunk = p_scrs[s][g]
            eqg = chunk == m
            repl = jnp.max(jnp.where(chunk < m, chunk, neg), axis=0,
                           keepdims=True)
            cellmask = gmask & (g_iota == g)
            wrow = jnp.where((lr_iota == 0) & ((g == 0) | (g == _NCH - 1)),
                             jnp.float32(1.0), jnp.float32(2.0))
            cnt_col = jnp.sum(jnp.where(eqg, wrow, jnp.float32(0.0)),
                              axis=0, keepdims=True)
            colsel = jnp.sum(cellmask.astype(jnp.float32), axis=0,
                             keepdims=True)
            w = jnp.sum(cnt_col * colsel, axis=(0, 1),
                        keepdims=True)
            msum = jnp.where(cellmask, repl, msum)
            val = jnp.sqrt(m)
            out = jnp.where((lane >= cnt) & (lane < cnt + w), val, out)
            new.append((msum, out, cnt + w))
        return tuple(new)

    carry = jax.lax.fori_loop(0, _K, body, tuple(carry))
    for s in range(_S):
        out_ref[s] = carry[s][1]


def kernel(mtrx):
    b, c, h, w = mtrx.shape
    x = mtrx.reshape(b * c, h, w).astype(jnp.bfloat16)
    out = pl.pallas_call(
        _fft_topk_kernel,
        grid=(b * c // _S,),
        in_specs=[
            pl.BlockSpec((_S, h, w), lambda i: (i, 0, 0)),
            pl.BlockSpec((_HP, _N), lambda i: (0, 0)),
            pl.BlockSpec((_HP, _N), lambda i: (0, 0)),
            pl.BlockSpec((_N, _N), lambda i: (0, 0)),
            pl.BlockSpec((_N, _N), lambda i: (0, 0)),
        ],
        out_specs=pl.BlockSpec((_S, 1, _K), lambda i: (i, 0, 0)),
        out_shape=jax.ShapeDtypeStruct((b * c, 1, _K), jnp.float32),
        scratch_shapes=[pltpu.VMEM((_NCH, _CH, _N), jnp.float32)
                        for _ in range(_S)],
        compiler_params=pltpu.CompilerParams(
            dimension_semantics=("arbitrary",)),
    )(x, jnp.asarray(_FRE_H), jnp.asarray(_FIM_H),
      jnp.asarray(_FRE), jnp.asarray(_FIM))
    return out.reshape(b, c, _K)


# speedup vs baseline: 1.8626x; 1.8626x over previous
import numpy as np
import jax
import jax.numpy as jnp
from jax.experimental import pallas as pl
from jax.experimental.pallas import tpu as pltpu

_N = 384
_H = 193
_HP = 208
_NCH = 13
_CH = 16
_K = 16
_S = 4
_BIG = np.int32(1 << 30)


def _dft_consts():
    j = np.arange(_N)
    m = np.outer(j, j) % _N
    ang = -2.0 * np.pi * m / _N
    fre = (np.cos(ang) / np.sqrt(_N)).astype(np.float32)
    fim = (np.sin(ang) / np.sqrt(_N)).astype(np.float32)
    fre_h = np.zeros((_HP, _N), np.float32)
    fim_h = np.zeros((_HP, _N), np.float32)
    fre_h[:_H] = fre[:_H]
    fim_h[:_H] = fim[:_H]
    import ml_dtypes
    bf = ml_dtypes.bfloat16
    return fre_h.astype(bf), fim_h.astype(bf), fre.astype(bf), fim.astype(bf)


_FRE_H, _FIM_H, _FRE, _FIM = _dft_consts()


def _fft_topk_kernel(x_ref, freh_ref, fimh_ref, fre_ref, fim_ref, out_ref,
                     *p_scrs):
    freh = freh_ref[...]
    fimh = fimh_ref[...]
    fre = fre_ref[...]
    fim = fim_ref[...]

    def dot(a, b):
        return jax.lax.dot(a, b, preferred_element_type=jnp.float32)

    neg = jnp.float32(-np.inf)
    msums = []
    for s in range(_S):
        x = x_ref[s]
        bre = dot(freh, x)
        bim = dot(fimh, x)
        breb = bre.astype(jnp.bfloat16)
        bimb = bim.astype(jnp.bfloat16)
        yre = dot(breb, fre) - dot(bimb, fim)
        yim = dot(breb, fim) + dot(bimb, fre)
        p = yre * yre + yim * yim

        mrows = [jnp.max(p[g * _CH:(g + 1) * _CH], axis=0, keepdims=True)
                 for g in range(_NCH)]
        mrows.append(jnp.full((16 - _NCH, _N), neg, jnp.float32))
        msums.append(jnp.concatenate(mrows, axis=0))
        p_scrs[s][...] = p.reshape(_NCH, _CH, _N)

    g_iota = jax.lax.broadcasted_iota(jnp.int32, (16, _N), 0)
    lr_iota = jax.lax.broadcasted_iota(jnp.int32, (_CH, _N), 0)
    lane = jax.lax.broadcasted_iota(jnp.int32, (1, _K), 1)

    def cond(carry):
        return sum([jnp.where(carry[2][s] < _K, 1, 0) for s in range(_S)]) > 0

    def body(carry):
        msum_l, out_l, cnt_l = carry
        nmsum, nout, ncnt = [], [], []
        for s in range(_S):
            msum, out, cnt = msum_l[s], out_l[s], cnt_l[s]
            m = jnp.max(msum)
            g = jnp.min(jnp.where(msum == m, g_iota, _BIG))
            chunk = p_scrs[s][g]
            eqg = chunk == m
            repl = jnp.max(jnp.where(chunk < m, chunk, neg), axis=0,
                           keepdims=True)
            cellmask = (msum == m) & (g_iota == g)
            wrow = jnp.where((lr_iota == 0) & ((g == 0) | (g == _NCH - 1)),
                             jnp.float32(1.0), jnp.float32(2.0))
            cnt_col = jnp.sum(jnp.where(eqg, wrow, jnp.float32(0.0)),
                              axis=0, keepdims=True)
            colsel = jnp.sum(cellmask.astype(jnp.float32), axis=0,
                             keepdims=True)
            w = jnp.sum(cnt_col * colsel).astype(jnp.int32)
            nmsum.append(jnp.where(cellmask, repl, msum))
            val = jnp.sqrt(m)
            nout.append(jnp.where((lane >= cnt) & (lane < cnt + w),
                                  val, out))
            ncnt.append(cnt + w)
        return tuple(nmsum), tuple(nout), tuple(ncnt)

    zero_out = tuple(jnp.zeros((1, _K), jnp.float32) for _ in range(_S))
    zero_cnt = tuple(jnp.int32(0) for _ in range(_S))
    _, out_l, _ = jax.lax.while_loop(
        cond, body, (tuple(msums), zero_out, zero_cnt))
    for s in range(_S):
        out_ref[s] = out_l[s]


def kernel(mtrx):
    b, c, h, w = mtrx.shape
    x = mtrx.reshape(b * c, h, w).astype(jnp.bfloat16)
    out = pl.pallas_call(
        _fft_topk_kernel,
        grid=(b * c // _S,),
        in_specs=[
            pl.BlockSpec((_S, h, w), lambda i: (i, 0, 0)),
            pl.BlockSpec((_HP, _N), lambda i: (0, 0)),
            pl.BlockSpec((_HP, _N), lambda i: (0, 0)),
            pl.BlockSpec((_N, _N), lambda i: (0, 0)),
            pl.BlockSpec((_N, _N), lambda i: (0, 0)),
        ],
        out_specs=pl.BlockSpec((_S, 1, _K), lambda i: (i, 0, 0)),
        out_shape=jax.ShapeDtypeStruct((b * c, 1, _K), jnp.float32),
        scratch_shapes=[pltpu.VMEM((_NCH, _CH, _N), jnp.float32)
                        for _ in range(_S)],
        compiler_params=pltpu.CompilerParams(
            dimension_semantics=("arbitrary",)),
    )(x, jnp.asarray(_FRE_H), jnp.asarray(_FIM_H),
      jnp.asarray(_FRE), jnp.asarray(_FIM))
    return out.reshape(b, c, _K)


# speedup vs baseline: 1.9209x; 1.0313x over previous
import numpy as np
import jax
import jax.numpy as jnp
from jax.experimental import pallas as pl
from jax.experimental.pallas import tpu as pltpu

_N = 384
_H = 193
_HP = 208
_NCH = 13
_CH = 16
_K = 16
_S = 8
_BIG = np.int32(1 << 30)


def _dft_consts():
    j = np.arange(_N)
    m = np.outer(j, j) % _N
    ang = -2.0 * np.pi * m / _N
    fre = (np.cos(ang) / np.sqrt(_N)).astype(np.float32)
    fim = (np.sin(ang) / np.sqrt(_N)).astype(np.float32)
    fre_h = np.zeros((_HP, _N), np.float32)
    fim_h = np.zeros((_HP, _N), np.float32)
    fre_h[:_H] = fre[:_H]
    fim_h[:_H] = fim[:_H]
    import ml_dtypes
    bf = ml_dtypes.bfloat16
    return fre_h.astype(bf), fim_h.astype(bf), fre.astype(bf), fim.astype(bf)


_FRE_H, _FIM_H, _FRE, _FIM = _dft_consts()


def _fft_topk_kernel(x_ref, freh_ref, fimh_ref, fre_ref, fim_ref, out_ref,
                     *p_scrs):
    freh = freh_ref[...]
    fimh = fimh_ref[...]
    fre = fre_ref[...]
    fim = fim_ref[...]

    def dot(a, b):
        return jax.lax.dot(a, b, preferred_element_type=jnp.float32)

    neg = jnp.float32(-np.inf)
    msums = []
    for s in range(_S):
        x = x_ref[s]
        bre = dot(freh, x)
        bim = dot(fimh, x)
        breb = bre.astype(jnp.bfloat16)
        bimb = bim.astype(jnp.bfloat16)
        yre = dot(breb, fre) - dot(bimb, fim)
        yim = dot(breb, fim) + dot(bimb, fre)
        p = yre * yre + yim * yim

        mrows = [jnp.max(p[g * _CH:(g + 1) * _CH], axis=0, keepdims=True)
                 for g in range(_NCH)]
        mrows.append(jnp.full((16 - _NCH, _N), neg, jnp.float32))
        msums.append(jnp.concatenate(mrows, axis=0))
        p_scrs[s][...] = p.reshape(_NCH, _CH, _N)

    g_iota = jax.lax.broadcasted_iota(jnp.int32, (16, _N), 0)
    lr_iota = jax.lax.broadcasted_iota(jnp.int32, (_CH, _N), 0)
    lane = jax.lax.broadcasted_iota(jnp.int32, (1, _K), 1)

    def cond(carry):
        return sum([jnp.where(carry[2][s] < _K, 1, 0) for s in range(_S)]) > 0

    def body(carry):
        msum_l, out_l, cnt_l = carry
        nmsum, nout, ncnt = [], [], []
        for s in range(_S):
            msum, out, cnt = msum_l[s], out_l[s], cnt_l[s]
            m = jnp.max(msum)
            g = jnp.min(jnp.where(msum == m, g_iota, _BIG))
            chunk = p_scrs[s][g]
            eqg = chunk == m
            repl = jnp.max(jnp.where(chunk < m, chunk, neg), axis=0,
                           keepdims=True)
            cellmask = (msum == m) & (g_iota == g)
            wrow = jnp.where((lr_iota == 0) & ((g == 0) | (g == _NCH - 1)),
                             jnp.float32(1.0), jnp.float32(2.0))
            cnt_col = jnp.sum(jnp.where(eqg, wrow, jnp.float32(0.0)),
                              axis=0, keepdims=True)
            colsel = jnp.sum(cellmask.astype(jnp.float32), axis=0,
                             keepdims=True)
            w = jnp.sum(cnt_col * colsel).astype(jnp.int32)
            nmsum.append(jnp.where(cellmask, repl, msum))
            val = jnp.sqrt(m)
            nout.append(jnp.where((lane >= cnt) & (lane < cnt + w),
                                  val, out))
            ncnt.append(cnt + w)
        return tuple(nmsum), tuple(nout), tuple(ncnt)

    zero_out = tuple(jnp.zeros((1, _K), jnp.float32) for _ in range(_S))
    zero_cnt = tuple(jnp.int32(0) for _ in range(_S))
    _, out_l, _ = jax.lax.while_loop(
        cond, body, (tuple(msums), zero_out, zero_cnt))
    for s in range(_S):
        out_ref[s] = out_l[s]


def kernel(mtrx):
    b, c, h, w = mtrx.shape
    x = mtrx.reshape(b * c, h, w).astype(jnp.bfloat16)
    out = pl.pallas_call(
        _fft_topk_kernel,
        grid=(b * c // _S,),
        in_specs=[
            pl.BlockSpec((_S, h, w), lambda i: (i, 0, 0)),
            pl.BlockSpec((_HP, _N), lambda i: (0, 0)),
            pl.BlockSpec((_HP, _N), lambda i: (0, 0)),
            pl.BlockSpec((_N, _N), lambda i: (0, 0)),
            pl.BlockSpec((_N, _N), lambda i: (0, 0)),
        ],
        out_specs=pl.BlockSpec((_S, 1, _K), lambda i: (i, 0, 0)),
        out_shape=jax.ShapeDtypeStruct((b * c, 1, _K), jnp.float32),
        scratch_shapes=[pltpu.VMEM((_NCH, _CH, _N), jnp.float32)
                        for _ in range(_S)],
        compiler_params=pltpu.CompilerParams(
            dimension_semantics=("arbitrary",)),
    )(x, jnp.asarray(_FRE_H), jnp.asarray(_FIM_H),
      jnp.asarray(_FRE), jnp.asarray(_FIM))
    return out.reshape(b, c, _K)


# speedup vs baseline: 1.9215x; 1.0003x over previous
import numpy as np
import jax
import jax.numpy as jnp
from jax.experimental import pallas as pl
from jax.experimental.pallas import tpu as pltpu

_N = 384
_H = 193
_HP = 208
_NCH = 13
_CH = 16
_K = 16
_S = 8
_BIG = np.int32(1 << 30)


def _dft_consts():
    j = np.arange(_N)
    m = np.outer(j, j) % _N
    ang = -2.0 * np.pi * m / _N
    fre = (np.cos(ang) / np.sqrt(_N)).astype(np.float32)
    fim = (np.sin(ang) / np.sqrt(_N)).astype(np.float32)
    fre_h = np.zeros((_HP, _N), np.float32)
    fim_h = np.zeros((_HP, _N), np.float32)
    fre_h[:_H] = fre[:_H]
    fim_h[:_H] = fim[:_H]
    import ml_dtypes
    bf = ml_dtypes.bfloat16
    return fre_h.astype(bf), fim_h.astype(bf), fre.astype(bf), fim.astype(bf)


_FRE_H, _FIM_H, _FRE, _FIM = _dft_consts()


def _fft_topk_kernel(x_ref, freh_ref, fimh_ref, fre_ref, fim_ref, out_ref,
                     *p_scrs):
    freh = freh_ref[...]
    fimh = fimh_ref[...]
    fre = fre_ref[...]
    fim = fim_ref[...]

    def dot(a, b):
        return jax.lax.dot(a, b, preferred_element_type=jnp.float32)

    neg = jnp.float32(-np.inf)
    msums = []
    for s in range(_S):
        x = x_ref[s]
        bre = dot(freh, x)
        bim = dot(fimh, x)
        breb = bre.astype(jnp.bfloat16)
        bimb = bim.astype(jnp.bfloat16)
        yre = dot(breb, fre) - dot(bimb, fim)
        yim = dot(breb, fim) + dot(bimb, fre)
        p = yre * yre + yim * yim

        mrows = [jnp.max(p[g * _CH:(g + 1) * _CH], axis=0, keepdims=True)
                 for g in range(_NCH)]
        mrows.append(jnp.full((16 - _NCH, _N), neg, jnp.float32))
        msums.append(jnp.concatenate(mrows, axis=0))
        p_scrs[s][...] = p.reshape(_NCH, _CH, _N)

    g_iota = jax.lax.broadcasted_iota(jnp.int32, (16, _N), 0)
    lr_iota = jax.lax.broadcasted_iota(jnp.int32, (_CH, _N), 0)
    lane = jax.lax.broadcasted_iota(jnp.int32, (1, _K), 1)

    def cond(carry):
        return sum([jnp.where(carry[2][s] < _K, 1, 0) for s in range(_S)]) > 0

    def body(carry):
        msum_l, out_l, cnt_l = carry
        nmsum, nout, ncnt = [], [], []
        for s in range(_S):
            msum, out, cnt = msum_l[s], out_l[s], cnt_l[s]
            m = jnp.max(msum)
            g = jnp.min(jnp.where(msum == m, g_iota, _BIG))
            chunk = p_scrs[s][g]
            eqg = chunk == m
            repl = jnp.max(jnp.where(chunk < m, chunk, neg), axis=0,
                           keepdims=True)
            cellmask = (msum == m) & (g_iota == g)
            wrow = jnp.where((lr_iota == 0) & ((g == 0) | (g == _NCH - 1)),
                             jnp.float32(1.0), jnp.float32(2.0))
            cnt_col = jnp.sum(jnp.where(eqg, wrow, jnp.float32(0.0)),
                              axis=0, keepdims=True)
            colsel = jnp.sum(cellmask.astype(jnp.float32), axis=0,
                             keepdims=True)
            w = jnp.sum(cnt_col * colsel).astype(jnp.int32)
            nmsum.append(jnp.where(cellmask, repl, msum))
            val = jnp.sqrt(m)
            nout.append(jnp.where((lane >= cnt) & (lane < cnt + w),
                                  val, out))
            ncnt.append(cnt + w)
        return tuple(nmsum), tuple(nout), tuple(ncnt)

    zero_out = tuple(jnp.zeros((1, _K), jnp.float32) for _ in range(_S))
    zero_cnt = tuple(jnp.int32(0) for _ in range(_S))
    _, out_l, _ = jax.lax.while_loop(
        cond, body, (tuple(msums), zero_out, zero_cnt))
    for s in range(_S):
        out_ref[s] = out_l[s]


def kernel(mtrx):
    b, c, h, w = mtrx.shape
    x = mtrx.reshape(b * c, h, w).astype(jnp.bfloat16)
    out = pl.pallas_call(
        _fft_topk_kernel,
        grid=(b * c // _S,),
        in_specs=[
            pl.BlockSpec((_S, h, w), lambda i: (i, 0, 0)),
            pl.BlockSpec((_HP, _N), lambda i: (0, 0)),
            pl.BlockSpec((_HP, _N), lambda i: (0, 0)),
            pl.BlockSpec((_N, _N), lambda i: (0, 0)),
            pl.BlockSpec((_N, _N), lambda i: (0, 0)),
        ],
        out_specs=pl.BlockSpec((_S, 1, _K), lambda i: (i, 0, 0)),
        out_shape=jax.ShapeDtypeStruct((b * c, 1, _K), jnp.float32),
        scratch_shapes=[pltpu.VMEM((_NCH, _CH, _N), jnp.float32)
                        for _ in range(_S)],
        compiler_params=pltpu.CompilerParams(
            dimension_semantics=("parallel",)),
    )(x, jnp.asarray(_FRE_H), jnp.asarray(_FIM_H),
      jnp.asarray(_FRE), jnp.asarray(_FIM))
    return out.reshape(b, c, _K)
